# baseline (speedup 1.0000x reference)
import jax
import jax.numpy as jnp
from jax.experimental import pallas as pl
from jax.experimental.pallas import tpu as pltpu


def kernel(x, table):
    B = x.shape[0]
    M, D = table.shape
    R = 1024

    def body(t_ref, o_ref):
        o_ref[...] = jnp.broadcast_to(t_ref[...][None], (B, R, D))

    return pl.pallas_call(
        body,
        grid=(M // R,),
        in_specs=[pl.BlockSpec((R, D), lambda i: (i, 0))],
        out_specs=pl.BlockSpec((B, R, D), lambda i: (0, i, 0)),
        out_shape=jax.ShapeDtypeStruct((B, M, D), table.dtype),
        compiler_params=pltpu.CompilerParams(
            dimension_semantics=("parallel",),
        ),
    )(table)
